# baseline (speedup 1.0000x reference)
import functools

import jax
import jax.numpy as jnp
from jax import lax
from jax.experimental import pallas as pl
from jax.experimental.pallas import tpu as pltpu
from jax.experimental.pallas import tpu_sc as plsc

_L = 16
_NW = 32
_P = 256


def _rsqrt16(s):
    s = jnp.maximum(s, 1e-24)
    i = lax.bitcast_convert_type(s, jnp.int32)
    i = jnp.int32(0x5F3759DF) - lax.shift_right_arithmetic(i, 1)
    y = lax.bitcast_convert_type(i, jnp.float32)
    for _ in range(3):
        y = y * (1.5 - 0.5 * s * y * y)
    return y


def _sc_partials(feats4):
    F, n, c, hw = feats4.shape
    px_total = n * hw
    px_w = px_total // _NW
    chunks = px_w // _P
    w_per_n = hw // px_w
    mesh = plsc.VectorSubcoreMesh(core_axis_name="c", subcore_axis_name="s")

    @functools.partial(
        pl.kernel,
        mesh=mesh,
        out_type=jax.ShapeDtypeStruct((_NW, _L), jnp.float32),
        scratch_types=[
            pltpu.VMEM((F, c, _P), jnp.float32),
            pltpu.VMEM((_L,), jnp.float32),
        ],
    )
    def k(feats_hbm, out_hbm, buf, accv):
        cid = lax.axis_index("c")
        sid = lax.axis_index("s")
        wid = sid * 2 + cid
        n0 = wid // w_per_n
        base = (wid % w_per_n) * px_w

        def chunk_body(ch, acc):
            p0 = base + ch * _P
            for f in range(F):
                pltpu.sync_copy(feats_hbm.at[f, n0, :, pl.ds(p0, _P)],
                                buf.at[f])
            for g in range(_P // _L):
                sl = pl.ds(g * _L, _L)
                rs = []
                for f in range(F):
                    def sq_body(i, s, f=f, sl=sl):
                        v = buf[f, i, sl]
                        return s + v * v
                    s = lax.fori_loop(0, c, sq_body,
                                      jnp.zeros((_L,), jnp.float32))
                    rs.append(_rsqrt16(s))

                def pair_body(i, a, sl=sl, rs=rs):
                    ys = [buf[f, i, sl] * rs[f] for f in range(F)]
                    for f in range(F - 1):
                        a = a + jnp.abs(ys[f] - ys[f + 1])
                    return a
                acc = lax.fori_loop(0, c, pair_body, acc)
            return acc

        acc = lax.fori_loop(0, chunks, chunk_body,
                            jnp.zeros((_L,), jnp.float32))
        accv[...] = acc
        pltpu.sync_copy(accv, out_hbm.at[wid])

    return k(feats4)


def kernel(feats, scores, masks):
    del scores, masks
    F, n, c, h, w = feats.shape
    scale = 1.0 / ((F - 1) * n * c * h * w)
    feats4 = feats.reshape(F, n, c, h * w)
    partials = _sc_partials(feats4)
    return jnp.sum(partials) * scale


# speedup vs baseline: 1.9176x; 1.9176x over previous
import functools

import jax
import jax.numpy as jnp
from jax import lax
from jax.experimental import pallas as pl
from jax.experimental.pallas import tpu as pltpu
from jax.experimental.pallas import tpu_sc as plsc

_L = 16
_NW = 32
_P = 128
_H_SC = 128


def _rsqrt16(s):
    s = jnp.maximum(s, 1e-24)
    i = lax.bitcast_convert_type(s, jnp.int32)
    i = jnp.int32(0x5F3759DF) - lax.shift_right_arithmetic(i, 1)
    y = lax.bitcast_convert_type(i, jnp.float32)
    for _ in range(3):
        y = y * (1.5 - 0.5 * s * y * y)
    return y


def _sc_partials(feats4, px_n):
    F, n, c, hw = feats4.shape
    px_w = n * px_n // _NW
    chunks = px_w // _P
    w_per_n = px_n // px_w
    mesh = plsc.VectorSubcoreMesh(core_axis_name="c", subcore_axis_name="s")

    @functools.partial(
        pl.kernel,
        mesh=mesh,
        out_type=jax.ShapeDtypeStruct((_NW, _L), jnp.float32),
        scratch_types=[
            pltpu.VMEM((2, F, c, _P), jnp.float32),
            pltpu.VMEM((_L,), jnp.float32),
            pltpu.SemaphoreType.DMA,
            pltpu.SemaphoreType.DMA,
        ],
    )
    def k(feats_hbm, out_hbm, buf, accv, sem0, sem1):
        cid = lax.axis_index("c")
        sid = lax.axis_index("s")
        wid = sid * 2 + cid
        n0 = wid // w_per_n
        base = (wid % w_per_n) * px_w
        sems = (sem0, sem1)

        def fire(ch, b):
            p0 = base + ch * _P
            for f in range(F):
                pltpu.make_async_copy(
                    feats_hbm.at[f, n0, :, pl.ds(p0, _P)],
                    buf.at[b, f], sems[b]).start()

        def drain(ch, b):
            p0 = base + ch * _P
            for f in range(F):
                pltpu.make_async_copy(
                    feats_hbm.at[f, n0, :, pl.ds(p0, _P)],
                    buf.at[b, f], sems[b]).wait()

        fire(0, 0)

        def chunk_body(ch, acc, b):
            drain(ch, b)

            @pl.when(ch + 1 < chunks)
            def _():
                fire(ch + 1, 1 - b)

            for g in range(_P // _L):
                sl = pl.ds(g * _L, _L)

                def sq_body(i, s, b=b, sl=sl):
                    vs = [buf[b, f, i, sl] for f in range(F)]
                    return tuple(s[f] + vs[f] * vs[f] for f in range(F))
                z = jnp.zeros((_L,), jnp.float32)
                ss = lax.fori_loop(0, c, sq_body, (z,) * F, unroll=8)
                rs = [_rsqrt16(s) for s in ss]

                def pair_body(i, a, b=b, sl=sl, rs=rs):
                    ys = [buf[b, f, i, sl] * rs[f] for f in range(F)]
                    for f in range(F - 1):
                        a = a + jnp.abs(ys[f] - ys[f + 1])
                    return a
                acc = lax.fori_loop(0, c, pair_body, acc, unroll=8)
            return acc

        def chunk2_body(ch2, acc):
            acc = chunk_body(ch2 * 2, acc, 0)
            acc = chunk_body(ch2 * 2 + 1, acc, 1)
            return acc

        acc = lax.fori_loop(0, chunks // 2, chunk2_body,
                            jnp.zeros((_L,), jnp.float32))
        accv[...] = acc
        pltpu.sync_copy(accv, out_hbm.at[wid])

    return k(feats4)


def _tc_body(x_ref, out_ref):
    i = pl.program_id(0)
    j = pl.program_id(1)

    @pl.when(jnp.logical_and(i == 0, j == 0))
    def _():
        out_ref[0, 0] = 0.0

    x = x_ref[...]
    s = jnp.sum(x * x, axis=2, keepdims=True)
    y = x * lax.rsqrt(jnp.maximum(s, 1e-24))
    d = jnp.abs(y[:-1] - y[1:])
    out_ref[0, 0] += jnp.sum(d)


def _tc_sum(feats, h_lo):
    F, n, c, h, w = feats.shape
    rows = h - h_lo
    hb = next(x for x in (64, 32, 16, 8) if rows % x == 0)
    n_h = rows // hb
    out = pl.pallas_call(
        _tc_body,
        grid=(n, n_h),
        in_specs=[
            pl.BlockSpec((F, 1, c, hb, w),
                         lambda i, j: (0, i, 0, j + h_lo // hb, 0)),
        ],
        out_specs=pl.BlockSpec(
            (1, 1), lambda i, j: (0, 0), memory_space=pltpu.SMEM
        ),
        out_shape=jax.ShapeDtypeStruct((1, 1), jnp.float32),
    )(feats)
    return out[0, 0]


def kernel(feats, scores, masks):
    del scores, masks
    F, n, c, h, w = feats.shape
    scale = 1.0 / ((F - 1) * n * c * h * w)
    total = jnp.float32(0.0)
    if _H_SC > 0:
        feats4 = feats.reshape(F, n, c, h * w)
        total = total + jnp.sum(_sc_partials(feats4, _H_SC * w))
    if _H_SC < h:
        total = total + _tc_sum(feats, _H_SC)
    return total * scale


# speedup vs baseline: 2.3163x; 1.2079x over previous
import functools

import jax
import jax.numpy as jnp
from jax import lax
from jax.experimental import pallas as pl
from jax.experimental.pallas import tpu as pltpu
from jax.experimental.pallas import tpu_sc as plsc

_L = 16
_NW = 32
_P = 128
_H_SC = 16


def _rsqrt16(s):
    s = jnp.maximum(s, 1e-24)
    i = lax.bitcast_convert_type(s, jnp.int32)
    i = jnp.int32(0x5F3759DF) - lax.shift_right_arithmetic(i, 1)
    y = lax.bitcast_convert_type(i, jnp.float32)
    for _ in range(3):
        y = y * (1.5 - 0.5 * s * y * y)
    return y


def _sc_partials(feats4, px_n):
    F, n, c, hw = feats4.shape
    px_w = n * px_n // _NW
    chunks = px_w // _P
    w_per_n = px_n // px_w
    mesh = plsc.VectorSubcoreMesh(core_axis_name="c", subcore_axis_name="s")

    @functools.partial(
        pl.kernel,
        mesh=mesh,
        out_type=jax.ShapeDtypeStruct((_NW, _L), jnp.float32),
        scratch_types=[
            pltpu.VMEM((2, F, c, _P), jnp.float32),
            pltpu.VMEM((_L,), jnp.float32),
            pltpu.SemaphoreType.DMA,
            pltpu.SemaphoreType.DMA,
        ],
    )
    def k(feats_hbm, out_hbm, buf, accv, sem0, sem1):
        cid = lax.axis_index("c")
        sid = lax.axis_index("s")
        wid = sid * 2 + cid
        n0 = wid // w_per_n
        base = (wid % w_per_n) * px_w
        sems = (sem0, sem1)

        def fire(ch, b):
            p0 = base + ch * _P
            for f in range(F):
                pltpu.make_async_copy(
                    feats_hbm.at[f, n0, :, pl.ds(p0, _P)],
                    buf.at[b, f], sems[b]).start()

        def drain(ch, b):
            p0 = base + ch * _P
            for f in range(F):
                pltpu.make_async_copy(
                    feats_hbm.at[f, n0, :, pl.ds(p0, _P)],
                    buf.at[b, f], sems[b]).wait()

        fire(0, 0)

        def chunk_body(ch, acc, b):
            drain(ch, b)

            @pl.when(ch + 1 < chunks)
            def _():
                fire(ch + 1, 1 - b)

            for g in range(_P // _L):
                sl = pl.ds(g * _L, _L)

                def sq_body(i, s, b=b, sl=sl):
                    vs = [buf[b, f, i, sl] for f in range(F)]
                    return tuple(s[f] + vs[f] * vs[f] for f in range(F))
                z = jnp.zeros((_L,), jnp.float32)
                ss = lax.fori_loop(0, c, sq_body, (z,) * F, unroll=8)
                rs = [_rsqrt16(s) for s in ss]

                def pair_body(i, a, b=b, sl=sl, rs=rs):
                    ys = [buf[b, f, i, sl] * rs[f] for f in range(F)]
                    for f in range(F - 1):
                        a = a + jnp.abs(ys[f] - ys[f + 1])
                    return a
                acc = lax.fori_loop(0, c, pair_body, acc, unroll=8)
            return acc

        def chunk2_body(ch2, acc):
            acc = chunk_body(ch2 * 2, acc, 0)
            acc = chunk_body(ch2 * 2 + 1, acc, 1)
            return acc

        acc = lax.fori_loop(0, chunks // 2, chunk2_body,
                            jnp.zeros((_L,), jnp.float32))
        accv[...] = acc
        pltpu.sync_copy(accv, out_hbm.at[wid])

    return k(feats4)


def _tc_body(x_ref, out_ref):
    i = pl.program_id(0)
    j = pl.program_id(1)

    @pl.when(jnp.logical_and(i == 0, j == 0))
    def _():
        out_ref[0, 0] = 0.0

    x = x_ref[...]
    s = jnp.sum(x * x, axis=2, keepdims=True)
    y = x * lax.rsqrt(jnp.maximum(s, 1e-24))
    d = jnp.abs(y[:-1] - y[1:])
    out_ref[0, 0] += jnp.sum(d)


def _tc_sum(feats, h_lo):
    F, n, c, h, w = feats.shape
    rows = h - h_lo
    hb = next(x for x in (64, 32, 16, 8) if rows % x == 0)
    n_h = rows // hb
    out = pl.pallas_call(
        _tc_body,
        grid=(n, n_h),
        in_specs=[
            pl.BlockSpec((F, 1, c, hb, w),
                         lambda i, j: (0, i, 0, j + h_lo // hb, 0)),
        ],
        out_specs=pl.BlockSpec(
            (1, 1), lambda i, j: (0, 0), memory_space=pltpu.SMEM
        ),
        out_shape=jax.ShapeDtypeStruct((1, 1), jnp.float32),
    )(feats)
    return out[0, 0]


def kernel(feats, scores, masks):
    del scores, masks
    F, n, c, h, w = feats.shape
    scale = 1.0 / ((F - 1) * n * c * h * w)
    total = jnp.float32(0.0)
    if _H_SC > 0:
        feats4 = feats.reshape(F, n, c, h * w)
        total = total + jnp.sum(_sc_partials(feats4, _H_SC * w))
    if _H_SC < h:
        total = total + _tc_sum(feats, _H_SC)
    return total * scale


# speedup vs baseline: 10.5470x; 4.5535x over previous
import jax
import jax.numpy as jnp
from jax import lax
from jax.experimental import pallas as pl
from jax.experimental.pallas import tpu as pltpu


def _body(x_ref, out_ref, *, scale):
    i = pl.program_id(0)
    j = pl.program_id(1)

    @pl.when(jnp.logical_and(i == 0, j == 0))
    def _():
        out_ref[0, 0] = 0.0

    x = x_ref[...]
    s = jnp.sum(x * x, axis=2, keepdims=True)
    y = x * lax.rsqrt(jnp.maximum(s, 1e-24))
    d = jnp.abs(y[:-1] - y[1:])
    out_ref[0, 0] += jnp.sum(d) * scale


def kernel(feats, scores, masks):
    del scores, masks
    F, n, c, h, w = feats.shape
    hb = next(x for x in (64, 32, 16, 8, 4, 2, 1) if h % x == 0)
    n_h = h // hb
    scale = 1.0 / ((F - 1) * n * c * h * w)

    out = pl.pallas_call(
        lambda x_ref, out_ref: _body(x_ref, out_ref, scale=scale),
        grid=(n, n_h),
        in_specs=[
            pl.BlockSpec((F, 1, c, hb, w), lambda i, j: (0, i, 0, j, 0)),
        ],
        out_specs=pl.BlockSpec(
            (1, 1), lambda i, j: (0, 0), memory_space=pltpu.SMEM
        ),
        out_shape=jax.ShapeDtypeStruct((1, 1), jnp.float32),
    )(feats)
    return out[0, 0]
